# baseline (speedup 1.0000x reference)
import jax
import jax.numpy as jnp
from jax.experimental import pallas as pl
from jax.experimental.pallas import tpu as pltpu

_TS = 26
_TOT = 126
_REP = 20
_F = _TS + (_TOT - _TS) * _REP
_D = 64
_B = 1024
_HB = _B // 2
_FB = 54
_PAIRS = _FB // 2
_NPAIR = _F // 2
_LANE = 128


def _body(xlo_ref, xhi_ref, tbl_ref, o_ref, spl_ref):
    @pl.when(pl.program_id(0) == 0)
    def _init():
        t = tbl_ref[...]
        spl_ref[...] = jnp.broadcast_to(t[:, :, None], (_TOT, _D, _LANE))

    i = pl.program_id(0)
    for j in range(_PAIRS):
        p = jnp.minimum(i * _PAIRS + j, _NPAIR - 1)
        rh = jnp.where(p < _TS // 2, p, _TS // 2 + (p - _TS // 2) % 50)
        pair = spl_ref[pl.ds(2 * rh, 2)]
        bias = jnp.concatenate([pair] * (_HB // _LANE), axis=2)
        o_ref[2 * j:2 * j + 2, :, 0:_HB] = xlo_ref[2 * j:2 * j + 2] + bias
        o_ref[2 * j:2 * j + 2, :, _HB:_B] = xhi_ref[2 * j:2 * j + 2] + bias


def kernel(X, table):
    x_t = jnp.transpose(X, (1, 2, 0))

    out = pl.pallas_call(
        _body,
        grid=(pl.cdiv(_F, _FB),),
        in_specs=[
            pl.BlockSpec((_FB, _D, _HB), lambda i: (i, 0, 0)),
            pl.BlockSpec((_FB, _D, _HB), lambda i: (i, 0, 1)),
            pl.BlockSpec((_TOT, _D), lambda i: (0, 0)),
        ],
        out_specs=pl.BlockSpec((_FB, _D, _B), lambda i: (i, 0, 0)),
        out_shape=jax.ShapeDtypeStruct((_F, _D, _B), X.dtype),
        scratch_shapes=[pltpu.VMEM((_TOT, _D, _LANE), jnp.float32)],
    )(x_t, x_t, table)
    return jnp.transpose(out, (2, 0, 1))
